# baseline (speedup 1.0000x reference)
import functools

import jax
import jax.numpy as jnp
from jax.experimental import pallas as pl
from jax.experimental.pallas import tpu as pltpu

_CLASSES = 80
_CONF = 0.05
_IOU = 0.5
_MAXDET = 100
_PRE = 256
_PBLK = 128


def _nms_block(y1_ref, x1_ref, y2_ref, x2_ref, sc_ref, keep_ref, sup_ref):
    y1 = y1_ref[...]
    x1 = x1_ref[...]
    y2 = y2_ref[...]
    x2 = x2_ref[...]
    sc = sc_ref[...]
    area = (y2 - y1) * (x2 - x1)
    valid = sc > _CONF
    sup_ref[...] = jnp.where(valid, 0.0, 1.0)
    row_id = jax.lax.broadcasted_iota(jnp.int32, (_PRE, _PBLK), 0)

    def body(i, _):
        y1i = y1_ref[pl.ds(i, 1), :]
        x1i = x1_ref[pl.ds(i, 1), :]
        y2i = y2_ref[pl.ds(i, 1), :]
        x2i = x2_ref[pl.ds(i, 1), :]
        supi = sup_ref[pl.ds(i, 1), :]
        area_i = (y2i - y1i) * (x2i - x1i)
        iy1 = jnp.maximum(y1i, y1)
        ix1 = jnp.maximum(x1i, x1)
        iy2 = jnp.minimum(y2i, y2)
        ix2 = jnp.minimum(x2i, x2)
        inter = jnp.maximum(iy2 - iy1, 0.0) * jnp.maximum(ix2 - ix1, 0.0)
        union = area_i + area - inter
        over = inter / jnp.maximum(union, 1e-8) > _IOU
        row = jnp.where(over & (row_id > i), 1.0, 0.0)
        sup = sup_ref[...]
        sup_ref[...] = jnp.maximum(sup, row * (1.0 - supi))
        return 0

    jax.lax.fori_loop(0, _PRE, body, 0, unroll=False)
    keep_ref[...] = jnp.where(valid & (sup_ref[...] < 0.5), sc, -1.0)


def _run_nms(y1t, x1t, y2t, x2t, sct):
    bp = y1t.shape[1]
    grid = (bp // _PBLK,)
    spec = pl.BlockSpec((_PRE, _PBLK), lambda i: (0, i))
    return pl.pallas_call(
        _nms_block,
        grid=grid,
        in_specs=[spec] * 5,
        out_specs=spec,
        out_shape=jax.ShapeDtypeStruct((_PRE, bp), jnp.float32),
        scratch_shapes=[pltpu.VMEM((_PRE, _PBLK), jnp.float32)],
    )(y1t, x1t, y2t, x2t, sct)


def kernel(predictions):
    B, N, _ = predictions.shape
    x1 = predictions[..., 0]
    y1 = predictions[..., 1]
    x2 = predictions[..., 2]
    y2 = predictions[..., 3]
    cls = predictions[..., 4].astype(jnp.int32)
    score = predictions[..., 5]

    bp = B * _CLASSES
    valid0 = score > _CONF
    cls_key = jnp.where(valid0, cls, 127)
    sbits = jax.lax.bitcast_convert_type(score, jnp.int32)
    neg_sbits = -jnp.where(valid0, sbits, 0)
    idx0 = jnp.broadcast_to(jnp.arange(N, dtype=jnp.int32)[None, :], (B, N))
    s_cls, _, s_idx = jax.lax.sort(
        (cls_key, neg_sbits, idx0), dimension=1, num_keys=3, is_stable=False
    )

    bidx = jnp.broadcast_to(jnp.arange(B, dtype=jnp.int32)[:, None], (B, N))
    cnt = (
        jnp.zeros((B, 128), jnp.int32)
        .at[bidx.reshape(-1), cls_key.reshape(-1)]
        .add(1)
    )[:, :_CLASSES]
    off = jnp.cumsum(cnt, axis=1) - cnt

    r_col = jnp.arange(_PRE, dtype=jnp.int32)[:, None]
    off_f = off.reshape(1, bp)
    cnt_f = cnt.reshape(1, bp)
    slot_ok = r_col < cnt_f
    pos = jnp.where(slot_ok, off_f + r_col, 0)
    bofs = (jnp.arange(bp, dtype=jnp.int32) // _CLASSES * N).reshape(1, bp)
    src = jnp.take(s_idx.reshape(-1), bofs + pos)
    top_scores_t = jnp.where(slot_ok, jnp.take(score.reshape(-1), bofs + src), 0.0)

    def _g(coord):
        return jnp.take(coord.reshape(-1), bofs + src)

    ty1 = _g(y1)
    tx1 = _g(x1)
    ty2 = _g(y2)
    tx2 = _g(x2)

    keep_t = _run_nms(ty1, tx1, ty2, tx2, top_scores_t)
    keep = keep_t.T.reshape(B, _CLASSES * _PRE)

    fin_scores, fin_idx = jax.lax.top_k(keep, _MAXDET)
    fc = fin_idx // _PRE
    fr = fin_idx % _PRE
    tpos = fr * bp + jnp.arange(B, dtype=jnp.int32)[:, None] * _CLASSES + fc
    fin_boxes = jnp.stack(
        [
            jnp.take(tx1.reshape(-1), tpos),
            jnp.take(ty1.reshape(-1), tpos),
            jnp.take(tx2.reshape(-1), tpos),
            jnp.take(ty2.reshape(-1), tpos),
        ],
        axis=-1,
    )
    fin_classes = fc.astype(jnp.float32)
    valid_mask = fin_scores > _CONF
    valid_detections = valid_mask.sum(axis=1).astype(jnp.int32)
    out = jnp.concatenate(
        [
            fin_boxes,
            fin_classes[..., None],
            jnp.maximum(fin_scores, 0.0)[..., None],
        ],
        axis=-1,
    )
    out = jnp.where(valid_mask[..., None], out, 0.0)
    return out, valid_detections


# speedup vs baseline: 1.1801x; 1.1801x over previous
import functools

import jax
import jax.numpy as jnp
from jax import lax
from jax.experimental import pallas as pl
from jax.experimental.pallas import tpu as pltpu
from jax.experimental.pallas import tpu_sc as plsc

_CLASSES = 80
_CONF = 0.05
_IOU = 0.5
_MAXDET = 100
_PRE = 256
_PBLK = 128


def _nms_block(y1_ref, x1_ref, y2_ref, x2_ref, sc_ref, keep_ref, sup_ref):
    y1 = y1_ref[...]
    x1 = x1_ref[...]
    y2 = y2_ref[...]
    x2 = x2_ref[...]
    sc = sc_ref[...]
    area = (y2 - y1) * (x2 - x1)
    valid = sc > _CONF
    sup_ref[...] = jnp.where(valid, 0.0, 1.0)
    row_id = jax.lax.broadcasted_iota(jnp.int32, (_PRE, _PBLK), 0)

    def body(i, _):
        y1i = y1_ref[pl.ds(i, 1), :]
        x1i = x1_ref[pl.ds(i, 1), :]
        y2i = y2_ref[pl.ds(i, 1), :]
        x2i = x2_ref[pl.ds(i, 1), :]
        supi = sup_ref[pl.ds(i, 1), :]
        area_i = (y2i - y1i) * (x2i - x1i)
        iy1 = jnp.maximum(y1i, y1)
        ix1 = jnp.maximum(x1i, x1)
        iy2 = jnp.minimum(y2i, y2)
        ix2 = jnp.minimum(x2i, x2)
        inter = jnp.maximum(iy2 - iy1, 0.0) * jnp.maximum(ix2 - ix1, 0.0)
        union = area_i + area - inter
        u = jnp.maximum(union, 1e-8)
        over = (inter - 0.5 * u) > u * 1.4901161193847656e-08
        row = jnp.where(over & (row_id > i), 1.0, 0.0)
        sup = sup_ref[...]
        sup_ref[...] = jnp.maximum(sup, row * (1.0 - supi))
        return 0

    jax.lax.fori_loop(0, _PRE, body, 0, unroll=False)
    keep_ref[...] = jnp.where(valid & (sup_ref[...] < 0.5), sc, -1.0)


def _run_nms(y1t, x1t, y2t, x2t, sct):
    bp = y1t.shape[1]
    grid = (bp // _PBLK,)
    spec = pl.BlockSpec((_PRE, _PBLK), lambda i: (0, i))
    return pl.pallas_call(
        _nms_block,
        grid=grid,
        in_specs=[spec] * 5,
        out_specs=spec,
        out_shape=jax.ShapeDtypeStruct((_PRE, bp), jnp.float32),
        scratch_shapes=[pltpu.VMEM((_PRE, _PBLK), jnp.float32)],
    )(y1t, x1t, y2t, x2t, sct)


_NW = 32


def _sc_extract(bp, nflat):
    total = _PRE * bp
    slab = total // _NW
    mesh = plsc.VectorSubcoreMesh(core_axis_name="c", subcore_axis_name="s")
    fvec = jax.ShapeDtypeStruct((total,), jnp.float32)

    @functools.partial(
        pl.kernel,
        mesh=mesh,
        out_type=[fvec] * 5,
        scratch_types=[
            pltpu.VMEM((slab,), jnp.int32),
            pltpu.VMEM((slab,), jnp.int32),
            pltpu.VMEM((bp,), jnp.int32),
            pltpu.VMEM((slab,), jnp.float32),
            pltpu.VMEM((slab,), jnp.float32),
            pltpu.VMEM((slab,), jnp.float32),
            pltpu.VMEM((slab,), jnp.float32),
            pltpu.VMEM((slab,), jnp.float32),
            pltpu.VMEM((slab,), jnp.float32),
            pltpu.SemaphoreType.DMA,
        ],
    )
    def k(
        gidx1_hbm,
        mask_hbm,
        bofs_hbm,
        sidx_hbm,
        sc_hbm,
        y1_hbm,
        x1_hbm,
        y2_hbm,
        x2_hbm,
        osc,
        oy1,
        ox1,
        oy2,
        ox2,
        gidx1_v,
        src_v,
        bofs_v,
        mask_v,
        sc_v,
        y1_v,
        x1_v,
        y2_v,
        x2_v,
        sem,
    ):
        wid = lax.axis_index("s") * 2 + lax.axis_index("c")
        base = wid * slab
        pltpu.sync_copy(gidx1_hbm.at[pl.ds(base, slab)], gidx1_v)
        pltpu.sync_copy(mask_hbm.at[pl.ds(base, slab)], mask_v)
        pltpu.sync_copy(bofs_hbm.at[pl.ds(0, bp)], bofs_v)
        pltpu.async_copy(sidx_hbm.at[gidx1_v], src_v, sem).wait()
        for row in range(slab // bp):
            for c in range(bp // 16):
                o = row * bp + c * 16
                src_v[pl.ds(o, 16)] = src_v[pl.ds(o, 16)] + bofs_v[pl.ds(c * 16, 16)]
        for hbm, vv in (
            (sc_hbm, sc_v),
            (y1_hbm, y1_v),
            (x1_hbm, x1_v),
            (y2_hbm, y2_v),
            (x2_hbm, x2_v),
        ):
            pltpu.async_copy(hbm.at[src_v], vv, sem).wait()
        for c in range(slab // 16):
            sc_v[pl.ds(c * 16, 16)] = sc_v[pl.ds(c * 16, 16)] * mask_v[pl.ds(c * 16, 16)]
        for out, vv in (
            (osc, sc_v),
            (oy1, y1_v),
            (ox1, x1_v),
            (oy2, y2_v),
            (ox2, x2_v),
        ):
            pltpu.sync_copy(vv, out.at[pl.ds(base, slab)])

    return k


def kernel(predictions):
    B, N, _ = predictions.shape
    x1 = predictions[..., 0]
    y1 = predictions[..., 1]
    x2 = predictions[..., 2]
    y2 = predictions[..., 3]
    cls = predictions[..., 4].astype(jnp.int32)
    score = predictions[..., 5]

    bp = B * _CLASSES
    valid0 = score > _CONF
    cls_key = jnp.where(valid0, cls, 127)
    sbits = jax.lax.bitcast_convert_type(score, jnp.int32)
    neg_sbits = -jnp.where(valid0, sbits, 0)
    idx0 = jnp.broadcast_to(jnp.arange(N, dtype=jnp.int32)[None, :], (B, N))
    s_cls, _, s_idx = jax.lax.sort(
        (cls_key, neg_sbits, idx0), dimension=1, num_keys=3, is_stable=False
    )

    bidx = jnp.broadcast_to(jnp.arange(B, dtype=jnp.int32)[:, None], (B, N))
    cnt = (
        jnp.zeros((B, 128), jnp.int32)
        .at[bidx.reshape(-1), cls_key.reshape(-1)]
        .add(1)
    )[:, :_CLASSES]
    off = jnp.cumsum(cnt, axis=1) - cnt

    r_col = jnp.arange(_PRE, dtype=jnp.int32)[:, None]
    off_f = off.reshape(1, bp)
    cnt_f = cnt.reshape(1, bp)
    slot_ok = r_col < cnt_f
    pos = jnp.where(slot_ok, off_f + r_col, r_col)
    bofs_row = jnp.arange(bp, dtype=jnp.int32) // _CLASSES * N
    gidx1 = (bofs_row[None, :] + pos).reshape(-1)
    tsc_f, ty1_f, tx1_f, ty2_f, tx2_f = _sc_extract(bp, B * N)(
        gidx1,
        slot_ok.astype(jnp.float32).reshape(-1),
        bofs_row,
        s_idx.reshape(-1),
        score.reshape(-1),
        y1.reshape(-1),
        x1.reshape(-1),
        y2.reshape(-1),
        x2.reshape(-1),
    )
    top_scores_t = tsc_f.reshape(_PRE, bp)
    ty1 = ty1_f.reshape(_PRE, bp)
    tx1 = tx1_f.reshape(_PRE, bp)
    ty2 = ty2_f.reshape(_PRE, bp)
    tx2 = tx2_f.reshape(_PRE, bp)

    keep_t = _run_nms(ty1, tx1, ty2, tx2, top_scores_t)
    keep = keep_t.T.reshape(B, _CLASSES * _PRE)

    fin_scores, fin_idx = jax.lax.top_k(keep, _MAXDET)
    fc = fin_idx // _PRE
    fr = fin_idx % _PRE
    tpos = fr * bp + jnp.arange(B, dtype=jnp.int32)[:, None] * _CLASSES + fc
    fin_boxes = jnp.stack(
        [
            jnp.take(tx1.reshape(-1), tpos),
            jnp.take(ty1.reshape(-1), tpos),
            jnp.take(tx2.reshape(-1), tpos),
            jnp.take(ty2.reshape(-1), tpos),
        ],
        axis=-1,
    )
    fin_classes = fc.astype(jnp.float32)
    valid_mask = fin_scores > _CONF
    valid_detections = valid_mask.sum(axis=1).astype(jnp.int32)
    out = jnp.concatenate(
        [
            fin_boxes,
            fin_classes[..., None],
            jnp.maximum(fin_scores, 0.0)[..., None],
        ],
        axis=-1,
    )
    out = jnp.where(valid_mask[..., None], out, 0.0)
    return out, valid_detections


# speedup vs baseline: 1.2171x; 1.0314x over previous
import functools

import jax
import jax.numpy as jnp
from jax import lax
from jax.experimental import pallas as pl
from jax.experimental.pallas import tpu as pltpu
from jax.experimental.pallas import tpu_sc as plsc

_CLASSES = 80
_CONF = 0.05
_IOU = 0.5
_MAXDET = 100
_PRE = 256
_PBLK = 128


def _nms_block(y1_ref, x1_ref, y2_ref, x2_ref, sc_ref, keep_ref, sup_ref):
    y1 = y1_ref[...]
    x1 = x1_ref[...]
    y2 = y2_ref[...]
    x2 = x2_ref[...]
    sc = sc_ref[...]
    area = (y2 - y1) * (x2 - x1)
    valid = sc > _CONF
    sup_ref[...] = jnp.where(valid, 0.0, 1.0)
    row_id = jax.lax.broadcasted_iota(jnp.int32, (_PRE, _PBLK), 0)
    iota8 = jax.lax.broadcasted_iota(jnp.int32, (8, _PBLK), 0)
    eps = 1.4901161193847656e-08

    def chunk_body(kk, _):
        g0 = kk * 8
        y1c = y1_ref[pl.ds(g0, 8), :]
        x1c = x1_ref[pl.ds(g0, 8), :]
        y2c = y2_ref[pl.ds(g0, 8), :]
        x2c = x2_ref[pl.ds(g0, 8), :]
        area_c = (y2c - y1c) * (x2c - x1c)
        supc = sup_ref[pl.ds(g0, 8), :]
        for i in range(8):
            iy1 = jnp.maximum(y1c[i : i + 1], y1c)
            ix1 = jnp.maximum(x1c[i : i + 1], x1c)
            iy2 = jnp.minimum(y2c[i : i + 1], y2c)
            ix2 = jnp.minimum(x2c[i : i + 1], x2c)
            inter = jnp.maximum(iy2 - iy1, 0.0) * jnp.maximum(ix2 - ix1, 0.0)
            u = jnp.maximum(area_c[i : i + 1] + area_c - inter, 1e-8)
            over = (inter - 0.5 * u) > u * eps
            rowf = jnp.where(over & (iota8 > i), 1.0, 0.0)
            supc = jnp.maximum(supc, rowf * (1.0 - supc[i : i + 1]))
        acc = jnp.zeros((_PRE, _PBLK), jnp.float32)
        for i in range(8):
            iy1 = jnp.maximum(y1c[i : i + 1], y1)
            ix1 = jnp.maximum(x1c[i : i + 1], x1)
            iy2 = jnp.minimum(y2c[i : i + 1], y2)
            ix2 = jnp.minimum(x2c[i : i + 1], x2)
            inter = jnp.maximum(iy2 - iy1, 0.0) * jnp.maximum(ix2 - ix1, 0.0)
            u = jnp.maximum(area_c[i : i + 1] + area - inter, 1e-8)
            over = (inter - 0.5 * u) > u * eps
            acc = jnp.maximum(
                acc, jnp.where(over, 1.0, 0.0) * (1.0 - supc[i : i + 1])
            )
        supall = jnp.maximum(
            sup_ref[...], jnp.where(row_id >= g0 + 8, acc, 0.0)
        )
        sup_ref[...] = supall
        sup_ref[pl.ds(g0, 8), :] = supc
        return 0

    jax.lax.fori_loop(0, _PRE // 8, chunk_body, 0, unroll=False)
    keep_ref[...] = jnp.where(valid & (sup_ref[...] < 0.5), sc, -1.0)


def _run_nms(y1t, x1t, y2t, x2t, sct):
    bp = y1t.shape[1]
    grid = (bp // _PBLK,)
    spec = pl.BlockSpec((_PRE, _PBLK), lambda i: (0, i))
    return pl.pallas_call(
        _nms_block,
        grid=grid,
        in_specs=[spec] * 5,
        out_specs=spec,
        out_shape=jax.ShapeDtypeStruct((_PRE, bp), jnp.float32),
        scratch_shapes=[pltpu.VMEM((_PRE, _PBLK), jnp.float32)],
    )(y1t, x1t, y2t, x2t, sct)


_NW = 32


def _sc_extract(bp, nflat):
    total = _PRE * bp
    slab = total // _NW
    nchunk16 = slab // 16
    mesh = plsc.VectorSubcoreMesh(core_axis_name="c", subcore_axis_name="s")
    fvec = jax.ShapeDtypeStruct((total,), jnp.float32)

    @functools.partial(
        pl.kernel,
        mesh=mesh,
        out_type=[fvec] * 5,
        scratch_types=[
            pltpu.VMEM((slab,), jnp.int32),
            pltpu.VMEM((slab,), jnp.int32),
            pltpu.VMEM((bp,), jnp.int32),
            pltpu.VMEM((slab,), jnp.float32),
            pltpu.VMEM((slab,), jnp.float32),
            pltpu.VMEM((slab,), jnp.float32),
            pltpu.VMEM((slab,), jnp.float32),
            pltpu.VMEM((slab,), jnp.float32),
            pltpu.VMEM((slab,), jnp.float32),
            pltpu.SemaphoreType.DMA,
        ],
    )
    def k(
        gidx1_hbm,
        mask_hbm,
        bofs_hbm,
        sidx_hbm,
        sc_hbm,
        y1_hbm,
        x1_hbm,
        y2_hbm,
        x2_hbm,
        osc,
        oy1,
        ox1,
        oy2,
        ox2,
        gidx1_v,
        src_v,
        bofs_v,
        mask_v,
        sc_v,
        y1_v,
        x1_v,
        y2_v,
        x2_v,
        sem,
    ):
        wid = lax.axis_index("s") * 2 + lax.axis_index("c")
        base = wid * slab
        pltpu.sync_copy(gidx1_hbm.at[pl.ds(base, slab)], gidx1_v)
        pltpu.sync_copy(mask_hbm.at[pl.ds(base, slab)], mask_v)
        pltpu.sync_copy(bofs_hbm.at[pl.ds(0, bp)], bofs_v)
        pltpu.async_copy(sidx_hbm.at[gidx1_v], src_v, sem).wait()
        nlane = bp // 16

        def add_bofs(j, _):
            c = j - (j // nlane) * nlane
            o = j * 16
            src_v[pl.ds(o, 16)] = src_v[pl.ds(o, 16)] + bofs_v[pl.ds(c * 16, 16)]
            return 0

        lax.fori_loop(0, nchunk16, add_bofs, 0, unroll=False)
        for hbm, vv in (
            (sc_hbm, sc_v),
            (y1_hbm, y1_v),
            (x1_hbm, x1_v),
            (y2_hbm, y2_v),
            (x2_hbm, x2_v),
        ):
            pltpu.async_copy(hbm.at[src_v], vv, sem).wait()

        def apply_mask(j, _):
            o = j * 16
            sc_v[pl.ds(o, 16)] = sc_v[pl.ds(o, 16)] * mask_v[pl.ds(o, 16)]
            return 0

        lax.fori_loop(0, nchunk16, apply_mask, 0, unroll=False)
        for out, vv in (
            (osc, sc_v),
            (oy1, y1_v),
            (ox1, x1_v),
            (oy2, y2_v),
            (ox2, x2_v),
        ):
            pltpu.sync_copy(vv, out.at[pl.ds(base, slab)])

    return k


def kernel(predictions):
    B, N, _ = predictions.shape
    x1 = predictions[..., 0]
    y1 = predictions[..., 1]
    x2 = predictions[..., 2]
    y2 = predictions[..., 3]
    cls = predictions[..., 4].astype(jnp.int32)
    score = predictions[..., 5]

    bp = B * _CLASSES
    valid0 = score > _CONF
    cls_key = jnp.where(valid0, cls, 127)
    sbits = jax.lax.bitcast_convert_type(score, jnp.int32)
    neg_sbits = -jnp.where(valid0, sbits, 0)
    idx0 = jnp.broadcast_to(jnp.arange(N, dtype=jnp.int32)[None, :], (B, N))
    s_cls, _, s_idx = jax.lax.sort(
        (cls_key, neg_sbits, idx0), dimension=1, num_keys=3, is_stable=False
    )

    bidx = jnp.broadcast_to(jnp.arange(B, dtype=jnp.int32)[:, None], (B, N))
    cnt = (
        jnp.zeros((B, 128), jnp.int32)
        .at[bidx.reshape(-1), cls_key.reshape(-1)]
        .add(1)
    )[:, :_CLASSES]
    off = jnp.cumsum(cnt, axis=1) - cnt

    r_col = jnp.arange(_PRE, dtype=jnp.int32)[:, None]
    off_f = off.reshape(1, bp)
    cnt_f = cnt.reshape(1, bp)
    slot_ok = r_col < cnt_f
    pos = jnp.where(slot_ok, off_f + r_col, r_col)
    bofs_row = jnp.arange(bp, dtype=jnp.int32) // _CLASSES * N
    gidx1 = (bofs_row[None, :] + pos).reshape(-1)
    tsc_f, ty1_f, tx1_f, ty2_f, tx2_f = _sc_extract(bp, B * N)(
        gidx1,
        slot_ok.astype(jnp.float32).reshape(-1),
        bofs_row,
        s_idx.reshape(-1),
        score.reshape(-1),
        y1.reshape(-1),
        x1.reshape(-1),
        y2.reshape(-1),
        x2.reshape(-1),
    )
    top_scores_t = tsc_f.reshape(_PRE, bp)
    ty1 = ty1_f.reshape(_PRE, bp)
    tx1 = tx1_f.reshape(_PRE, bp)
    ty2 = ty2_f.reshape(_PRE, bp)
    tx2 = tx2_f.reshape(_PRE, bp)

    keep_t = _run_nms(ty1, tx1, ty2, tx2, top_scores_t)
    keep = keep_t.T.reshape(B, _CLASSES * _PRE)

    fin_scores, fin_idx = jax.lax.top_k(keep, _MAXDET)
    fc = fin_idx // _PRE
    fr = fin_idx % _PRE
    tpos = fr * bp + jnp.arange(B, dtype=jnp.int32)[:, None] * _CLASSES + fc
    fin_boxes = jnp.stack(
        [
            jnp.take(tx1.reshape(-1), tpos),
            jnp.take(ty1.reshape(-1), tpos),
            jnp.take(tx2.reshape(-1), tpos),
            jnp.take(ty2.reshape(-1), tpos),
        ],
        axis=-1,
    )
    fin_classes = fc.astype(jnp.float32)
    valid_mask = fin_scores > _CONF
    valid_detections = valid_mask.sum(axis=1).astype(jnp.int32)
    out = jnp.concatenate(
        [
            fin_boxes,
            fin_classes[..., None],
            jnp.maximum(fin_scores, 0.0)[..., None],
        ],
        axis=-1,
    )
    out = jnp.where(valid_mask[..., None], out, 0.0)
    return out, valid_detections
